# baseline (speedup 1.0000x reference)
import functools

import jax
import jax.numpy as jnp
from jax import lax
from jax.experimental import pallas as pl
from jax.experimental.pallas import tpu as pltpu
from jax.experimental.pallas import tpu_sc as plsc

NW = 32
NB = 2
SB_TOTAL = 200 * 16
SB_PER_W = SB_TOTAL // NW


def _emb_body(x5_hbm, w_hbm, out5_hbm,
              idxv, widebuf, outbuf,
              gsem0, gsem1, ssem0, ssem1):
    wid = lax.axis_index("s") * 2 + lax.axis_index("c")
    iota16 = lax.iota(jnp.int32, 16)
    gsems = (gsem0, gsem1)
    ssems = (ssem0, ssem1)
    rvecs = [iota16 + 16 * g for g in range(8)]
    bvecs = [jnp.full((16,), b, jnp.int32) for b in range(NB)]

    def coords(t):
        s = wid * SB_PER_W + t
        j = s // 16
        i2 = s % 16
        return j // 8, j % 8, j, i2

    def stage_a(t, slot):
        jr, jj, j, i2 = coords(t)
        pltpu.sync_copy(x5_hbm.at[jr, pl.ds(NB * i2, NB), jj], idxv.at[slot])
        for b in range(NB):
            pltpu.async_copy(w_hbm.at[idxv.at[slot, b]],
                             widebuf.at[slot, b], gsems[slot])

    def stage_b(t, slot):
        jr, jj, j, i2 = coords(t)
        for b in range(NB):
            pltpu.make_async_copy(w_hbm.at[idxv.at[slot, b]],
                                  widebuf.at[slot, b], gsems[slot]).wait()

        @pl.when(t >= 2)
        def _():
            for b in range(NB):
                pltpu.make_async_copy(outbuf.at[slot, b],
                                      out5_hbm.at[j, :, NB * i2 + b],
                                      ssems[slot]).wait()

        wide = widebuf.at[slot]

        def diag(d0, carry):
            cvec = (iota16 + d0) & 63
            dmaj = lax.shift_right_logical(cvec, 3)
            dmin = cvec & 7
            for b in range(NB):
                for g in range(8):
                    vals = plsc.load_gather(wide, [bvecs[b], rvecs[g], cvec])
                    plsc.store_scatter(outbuf.at[slot, b],
                                       [dmaj, dmin, rvecs[g]], vals)
            return carry

        lax.fori_loop(0, 64, diag, 0)

        for b in range(NB):
            pltpu.async_copy(outbuf.at[slot, b],
                             out5_hbm.at[j, :, NB * i2 + b],
                             ssems[slot])

    stage_a(0, 0)

    def body(tt, carry):
        t0 = 2 * tt
        stage_a(t0 + 1, 1)
        stage_b(t0, 0)

        @pl.when(tt < SB_PER_W // 2 - 1)
        def _():
            stage_a(t0 + 2, 0)

        stage_b(t0 + 1, 1)
        return carry

    lax.fori_loop(0, SB_PER_W // 2, body, 0)

    for slot in range(2):
        _, _, j, i2 = coords(SB_PER_W - 2 + slot)
        for b in range(NB):
            pltpu.make_async_copy(outbuf.at[slot, b],
                                  out5_hbm.at[j, :, NB * i2 + b],
                                  ssems[slot]).wait()


@jax.jit
def _embedding_lookup(x5, weight):
    mesh = plsc.VectorSubcoreMesh(core_axis_name="c", subcore_axis_name="s")
    k = functools.partial(
        pl.kernel,
        mesh=mesh,
        out_type=jax.ShapeDtypeStruct((200, 8, 32, 8, 128), jnp.float32),
        scratch_types=[
            pltpu.VMEM((2, NB, 128), jnp.int32),
            pltpu.VMEM((2, NB, 128, 64), jnp.float32),
            pltpu.VMEM((2, NB, 8, 8, 128), jnp.float32),
            pltpu.SemaphoreType.DMA,
            pltpu.SemaphoreType.DMA,
            pltpu.SemaphoreType.DMA,
            pltpu.SemaphoreType.DMA,
        ],
        compiler_params=pltpu.CompilerParams(
            use_tc_tiling_on_sc=False, needs_layout_passes=False),
    )(_emb_body)
    return k(x5, weight)


def kernel(x, weight):
    x5 = x.reshape(32, 128, 25, 8).transpose(2, 0, 3, 1)
    out5 = _embedding_lookup(x5, weight)
    return out5.transpose(2, 4, 0, 1, 3).reshape(4096, 200, 64)


# speedup vs baseline: 1.0077x; 1.0077x over previous
import functools

import jax
import jax.numpy as jnp
from jax import lax
from jax.experimental import pallas as pl
from jax.experimental.pallas import tpu as pltpu
from jax.experimental.pallas import tpu_sc as plsc

NW = 32
NB = 2
SB_TOTAL = 200 * 16
SB_PER_W = SB_TOTAL // NW


def _emb_body(x5_hbm, w_hbm, out5_hbm,
              idxv, widebuf, outbuf,
              gsem0, gsem1, ssem0, ssem1):
    wid = lax.axis_index("s") * 2 + lax.axis_index("c")
    iota16 = lax.iota(jnp.int32, 16)
    gsems = (gsem0, gsem1)
    ssems = (ssem0, ssem1)
    rvecs = [iota16 + 16 * g for g in range(8)]
    bvecs = [jnp.full((16,), b, jnp.int32) for b in range(NB)]

    def coords(t):
        s = wid * SB_PER_W + t
        j = s // 16
        i2 = s % 16
        return j // 8, j % 8, j, i2

    def stage_a(t, slot):
        jr, jj, j, i2 = coords(t)
        pltpu.sync_copy(x5_hbm.at[jr, pl.ds(NB * i2, NB), jj], idxv.at[slot])
        for b in range(NB):
            pltpu.async_copy(w_hbm.at[idxv.at[slot, b]],
                             widebuf.at[slot, b], gsems[slot])

    def stage_b(t, slot):
        jr, jj, j, i2 = coords(t)
        for b in range(NB):
            pltpu.make_async_copy(w_hbm.at[idxv.at[slot, b]],
                                  widebuf.at[slot, b], gsems[slot]).wait()

        @pl.when(t >= 2)
        def _():
            for b in range(NB):
                pltpu.make_async_copy(outbuf.at[slot, b],
                                      out5_hbm.at[j, :, NB * i2 + b],
                                      ssems[slot]).wait()

        wide = widebuf.at[slot]

        def diag(d4, carry):
            for dk in range(4):
                cvec = (iota16 + (d4 * 4 + dk)) & 63
                dmaj = lax.shift_right_logical(cvec, 3)
                dmin = cvec & 7
                for b in range(NB):
                    for g in range(8):
                        vals = plsc.load_gather(
                            wide, [bvecs[b], rvecs[g], cvec])
                        plsc.store_scatter(outbuf.at[slot, b],
                                           [dmaj, dmin, rvecs[g]], vals)
            return carry

        lax.fori_loop(0, 16, diag, 0)

        for b in range(NB):
            pltpu.async_copy(outbuf.at[slot, b],
                             out5_hbm.at[j, :, NB * i2 + b],
                             ssems[slot])

    stage_a(0, 0)

    def body(tt, carry):
        t0 = 2 * tt
        stage_a(t0 + 1, 1)
        stage_b(t0, 0)

        @pl.when(tt < SB_PER_W // 2 - 1)
        def _():
            stage_a(t0 + 2, 0)

        stage_b(t0 + 1, 1)
        return carry

    lax.fori_loop(0, SB_PER_W // 2, body, 0)

    for slot in range(2):
        _, _, j, i2 = coords(SB_PER_W - 2 + slot)
        for b in range(NB):
            pltpu.make_async_copy(outbuf.at[slot, b],
                                  out5_hbm.at[j, :, NB * i2 + b],
                                  ssems[slot]).wait()


@jax.jit
def _embedding_lookup(x5, weight):
    mesh = plsc.VectorSubcoreMesh(core_axis_name="c", subcore_axis_name="s")
    k = functools.partial(
        pl.kernel,
        mesh=mesh,
        out_type=jax.ShapeDtypeStruct((200, 8, 32, 8, 128), jnp.float32),
        scratch_types=[
            pltpu.VMEM((2, NB, 128), jnp.int32),
            pltpu.VMEM((2, NB, 128, 64), jnp.float32),
            pltpu.VMEM((2, NB, 8, 8, 128), jnp.float32),
            pltpu.SemaphoreType.DMA,
            pltpu.SemaphoreType.DMA,
            pltpu.SemaphoreType.DMA,
            pltpu.SemaphoreType.DMA,
        ],
        compiler_params=pltpu.CompilerParams(
            use_tc_tiling_on_sc=False, needs_layout_passes=False),
    )(_emb_body)
    return k(x5, weight)


def kernel(x, weight):
    x5 = x.reshape(32, 128, 25, 8).transpose(2, 0, 3, 1)
    out5 = _embedding_lookup(x5, weight)
    return out5.transpose(2, 4, 0, 1, 3).reshape(4096, 200, 64)


# speedup vs baseline: 1.0666x; 1.0585x over previous
import functools

import jax
import jax.numpy as jnp
from jax import lax
from jax.experimental import pallas as pl
from jax.experimental.pallas import tpu as pltpu
from jax.experimental.pallas import tpu_sc as plsc

NW = 32
NB = 2
SB_TOTAL = 200 * 16
SB_PER_W = SB_TOTAL // NW


def _emb_body(x5_hbm, w_hbm, out5_hbm,
              idxv, widebuf, outbuf,
              gsem0, gsem1, ssem0, ssem1):
    wid = lax.axis_index("s") * 2 + lax.axis_index("c")
    iota16 = lax.iota(jnp.int32, 16)
    gsems = (gsem0, gsem1)
    ssems = (ssem0, ssem1)
    rvecs = [iota16 + 16 * g for g in range(8)]
    bvecs = [jnp.full((16,), b, jnp.int32) for b in range(NB)]

    def coords(t):
        s = wid * SB_PER_W + t
        j = s // 16
        i2 = s % 16
        return j // 8, j % 8, j, i2

    def stage_a(t, slot):
        jr, jj, j, i2 = coords(t)
        pltpu.sync_copy(x5_hbm.at[jr, pl.ds(NB * i2, NB), jj], idxv.at[slot])
        for b in range(NB):
            pltpu.async_copy(w_hbm.at[idxv.at[slot, b]],
                             widebuf.at[slot, b], gsems[slot])

    def stage_b(t, slot):
        jr, jj, j, i2 = coords(t)
        for b in range(NB):
            pltpu.make_async_copy(w_hbm.at[idxv.at[slot, b]],
                                  widebuf.at[slot, b], gsems[slot]).wait()

        @pl.when(t >= 2)
        def _():
            for b in range(NB):
                pltpu.make_async_copy(outbuf.at[slot, b],
                                      out5_hbm.at[j, :, NB * i2 + b],
                                      ssems[slot]).wait()

        wide = widebuf.at[slot]

        def diag(d4, carry):
            for dk in range(4):
                cvec = (iota16 + (d4 * 4 + dk)) & 63
                dmaj = lax.shift_right_logical(cvec, 3)
                dmin = cvec & 7
                for b in range(NB):
                    for g in range(8):
                        vals = plsc.load_gather(
                            wide, [bvecs[b], rvecs[g], cvec])
                        plsc.store_scatter(outbuf.at[slot, b],
                                           [dmaj, dmin, rvecs[g]], vals)
            return carry

        lax.fori_loop(0, 16, diag, 0)

        for b in range(NB):
            pltpu.async_copy(outbuf.at[slot, b],
                             out5_hbm.at[j, :, NB * i2 + b],
                             ssems[slot])

    stage_a(0, 0)

    def body(tt, carry):
        t0 = 2 * tt
        stage_a(t0 + 1, 1)
        stage_b(t0, 0)

        @pl.when(tt < SB_PER_W // 2 - 1)
        def _():
            stage_a(t0 + 2, 0)

        stage_b(t0 + 1, 1)
        return carry

    lax.fori_loop(0, SB_PER_W // 2, body, 0)

    for slot in range(2):
        _, _, j, i2 = coords(SB_PER_W - 2 + slot)
        for b in range(NB):
            pltpu.make_async_copy(outbuf.at[slot, b],
                                  out5_hbm.at[j, :, NB * i2 + b],
                                  ssems[slot]).wait()


@jax.jit
def _embedding_lookup(x5, weight):
    mesh = plsc.VectorSubcoreMesh(core_axis_name="c", subcore_axis_name="s")
    k = functools.partial(
        pl.kernel,
        mesh=mesh,
        out_type=jax.ShapeDtypeStruct((200, 8, 32, 8, 128), jnp.float32),
        scratch_types=[
            pltpu.VMEM((2, NB, 128), jnp.int32),
            pltpu.VMEM((2, NB, 128, 128), jnp.float32),
            pltpu.VMEM((2, NB, 8, 8, 128), jnp.float32),
            pltpu.SemaphoreType.DMA,
            pltpu.SemaphoreType.DMA,
            pltpu.SemaphoreType.DMA,
            pltpu.SemaphoreType.DMA,
        ],
        compiler_params=pltpu.CompilerParams(
            use_tc_tiling_on_sc=False, needs_layout_passes=False),
    )(_emb_body)
    return k(x5, weight)


def kernel(x, weight):
    x5 = x.reshape(32, 128, 25, 8).transpose(2, 0, 3, 1)
    wpad = jnp.pad(weight, ((0, 0), (0, 64)))
    out5 = _embedding_lookup(x5, wpad)
    return out5.transpose(2, 4, 0, 1, 3).reshape(4096, 200, 64)


# speedup vs baseline: 1.3871x; 1.3005x over previous
import functools

import jax
import jax.numpy as jnp
from jax import lax
from jax.experimental import pallas as pl
from jax.experimental.pallas import tpu as pltpu
from jax.experimental.pallas import tpu_sc as plsc

NW = 32
NB = 2
SB_TOTAL = 200 * 16
SB_PER_W = SB_TOTAL // NW


def _emb_body(x5_hbm, w_hbm, out5_hbm,
              idxv, widebuf, outbuf,
              gsem0, gsem1, ssem0, ssem1):
    wid = lax.axis_index("s") * 2 + lax.axis_index("c")
    iota16 = lax.iota(jnp.int32, 16)
    gsems = (gsem0, gsem1)
    ssems = (ssem0, ssem1)
    rvecs = [iota16 + 16 * g for g in range(8)]
    bvecs = [jnp.full((16,), b, jnp.int32) for b in range(NB)]

    def coords(t):
        s = wid * SB_PER_W + t
        j = s // 16
        i2 = s % 16
        return j // 8, j % 8, j, i2

    def stage_a(t, slot):
        jr, jj, j, i2 = coords(t)
        pltpu.sync_copy(x5_hbm.at[jr, pl.ds(NB * i2, NB), jj], idxv.at[slot])
        for b in range(NB):
            pltpu.async_copy(w_hbm.at[idxv.at[slot, b]],
                             widebuf.at[slot, b], gsems[slot])

    def stage_b(t, slot):
        jr, jj, j, i2 = coords(t)
        for b in range(NB):
            pltpu.make_async_copy(w_hbm.at[idxv.at[slot, b]],
                                  widebuf.at[slot, b], gsems[slot]).wait()

        @pl.when(t >= 2)
        def _():
            for b in range(NB):
                pltpu.make_async_copy(outbuf.at[slot, b],
                                      out5_hbm.at[j, :, NB * i2 + b],
                                      ssems[slot]).wait()

        wide = widebuf.at[slot]

        def diag(d4, carry):
            for dk in range(4):
                cvec = (iota16 + (d4 * 4 + dk)) & 63
                dmaj = lax.shift_right_logical(cvec, 3)
                dmin = cvec & 7
                for b in range(NB):
                    vals = [plsc.load_gather(wide, [bvecs[b], rvecs[g], cvec])
                            for g in range(8)]
                    for g in range(8):
                        plsc.store_scatter(outbuf.at[slot, b],
                                           [dmaj, dmin, rvecs[g]], vals[g])
            return carry

        lax.fori_loop(0, 16, diag, 0)

        for b in range(NB):
            pltpu.async_copy(outbuf.at[slot, b],
                             out5_hbm.at[j, :, NB * i2 + b],
                             ssems[slot])

    stage_a(0, 0)

    def body(tt, carry):
        t0 = 2 * tt
        stage_a(t0 + 1, 1)
        stage_b(t0, 0)

        @pl.when(tt < SB_PER_W // 2 - 1)
        def _():
            stage_a(t0 + 2, 0)

        stage_b(t0 + 1, 1)
        return carry

    lax.fori_loop(0, SB_PER_W // 2, body, 0)

    for slot in range(2):
        _, _, j, i2 = coords(SB_PER_W - 2 + slot)
        for b in range(NB):
            pltpu.make_async_copy(outbuf.at[slot, b],
                                  out5_hbm.at[j, :, NB * i2 + b],
                                  ssems[slot]).wait()


@jax.jit
def _embedding_lookup(x5, weight):
    mesh = plsc.VectorSubcoreMesh(core_axis_name="c", subcore_axis_name="s")
    k = functools.partial(
        pl.kernel,
        mesh=mesh,
        out_type=jax.ShapeDtypeStruct((200, 8, 32, 8, 128), jnp.float32),
        scratch_types=[
            pltpu.VMEM((2, NB, 128), jnp.int32),
            pltpu.VMEM((2, NB, 128, 128), jnp.float32),
            pltpu.VMEM((2, NB, 8, 8, 128), jnp.float32),
            pltpu.SemaphoreType.DMA,
            pltpu.SemaphoreType.DMA,
            pltpu.SemaphoreType.DMA,
            pltpu.SemaphoreType.DMA,
        ],
        compiler_params=pltpu.CompilerParams(
            use_tc_tiling_on_sc=False, needs_layout_passes=False),
    )(_emb_body)
    return k(x5, weight)


def kernel(x, weight):
    x5 = x.reshape(32, 128, 25, 8).transpose(2, 0, 3, 1)
    wpad = jnp.pad(weight, ((0, 0), (0, 64)))
    out5 = _embedding_lookup(x5, wpad)
    return out5.transpose(2, 4, 0, 1, 3).reshape(4096, 200, 64)
